# baseline (speedup 1.0000x reference)
import jax
import jax.numpy as jnp
from jax import lax
from jax.experimental import pallas as pl
from jax.experimental.pallas import tpu as pltpu

_THR = 0.5
_K = 300
_SLOTR = 3
_NCH = 160
_G = 50


def _topk_kern(x_ref, s_out, lab_out, sel_out, c_ref):
    C = 15
    ROWS = C * _NCH
    GR = ROWS // _G
    x = x_ref[0]
    c = jnp.where(x > _THR, x, 0.0)
    c_ref[...] = c
    cm0 = jnp.concatenate(
        [jnp.max(c[g * GR:(g + 1) * GR], axis=0, keepdims=True) for g in range(_G)],
        axis=0)
    slot_i = (lax.broadcasted_iota(jnp.int32, (_SLOTR, 128), 0) * 128
              + lax.broadcasted_iota(jnp.int32, (_SLOTR, 128), 1))
    g_i = lax.broadcasted_iota(jnp.int32, (_G, 128), 0)
    fl_i = (lax.broadcasted_iota(jnp.int32, (GR, 128), 0) * 128
            + lax.broadcasted_iota(jnp.int32, (GR, 128), 1))
    big = jnp.int32(2 ** 30)

    def body(i, car):
        cm, os_, ol, on = car
        m = jnp.max(cm)
        valid = m > _THR
        gstar = jnp.min(jnp.where(cm == m, g_i, big))
        start = gstar * GR
        blk = c_ref[pl.ds(start, GR), :]
        fl = jnp.min(jnp.where(blk == m, fl_i, big))
        rloc = fl // 128
        lstar = fl - rloc * 128
        nblk = jnp.where(fl_i == fl, 0.0, blk)
        c_ref[pl.ds(start, GR), :] = nblk
        nmax = jnp.max(nblk, axis=0, keepdims=True)
        cm = jnp.where(g_i == gstar, nmax, cm)
        r = start + rloc
        cls_idx = r // _NCH
        n = (r - cls_idx * _NCH) * 128 + lstar
        upd = (slot_i == i) & valid
        os_ = jnp.where(upd, m, os_)
        ol = jnp.where(upd, cls_idx, ol)
        on = jnp.where(upd, n, on)
        return (cm, os_, ol, on)

    init = (cm0,
            jnp.full((_SLOTR, 128), -1.0, jnp.float32),
            jnp.full((_SLOTR, 128), -1, jnp.int32),
            jnp.full((_SLOTR, 128), -1, jnp.int32))
    cm, os_, ol, on = lax.fori_loop(0, _K, body, init)
    s_out[0] = os_
    lab_out[0] = ol
    sel_out[0] = on


def _gather_kern(sel_ref, sc_ref, tab_ref, rot_ref, out_ref, roto, cfo, cfs):
    sum3 = ((lax.broadcasted_iota(jnp.int32, (45, 15), 0) // 3)
            == lax.broadcasted_iota(jnp.int32, (45, 15), 1)).astype(jnp.float32)

    def body(r, _):
        n = sel_ref[0, 0, r]
        s = sc_ref[0, 0, r]
        valid = s > _THR
        nsafe = jnp.maximum(n, 0)
        row = tab_ref[0, pl.ds(nsafe, 1), :]
        out_ref[0, pl.ds(r, 1), :] = jnp.where(valid, row, -1.0)
        rrow = rot_ref[0, pl.ds(nsafe, 1), :]
        roto[0, pl.ds(r, 1), :] = jnp.where(valid, rrow, -1.0)
        cfs[pl.ds(r, 1), :] = jnp.where(valid, jnp.abs(row[:, 63:108]), -1.0 / 3.0)
        return 0

    lax.fori_loop(0, 304, body, 0)
    cfo[0] = jnp.dot(cfs[...], sum3, preferred_element_type=jnp.float32)


def kernel(boxes3D, classification, locations, translation, rotation, confidence):
    B, N, C = classification.shape
    ROWS = C * _NCH
    NPAD = _NCH * 128

    cls_t = jnp.transpose(classification, (0, 2, 1))
    cls_t = jnp.pad(cls_t, ((0, 0), (0, 0), (0, NPAD - N)))
    cls3 = cls_t.reshape(B, ROWS, 128)

    scores_r, labels_r, sel_r = pl.pallas_call(
        _topk_kern,
        grid=(B,),
        in_specs=[pl.BlockSpec((1, ROWS, 128), lambda b: (b, 0, 0))],
        out_specs=[pl.BlockSpec((1, _SLOTR, 128), lambda b: (b, 0, 0))] * 3,
        out_shape=[
            jax.ShapeDtypeStruct((B, _SLOTR, 128), jnp.float32),
            jax.ShapeDtypeStruct((B, _SLOTR, 128), jnp.int32),
            jax.ShapeDtypeStruct((B, _SLOTR, 128), jnp.int32),
        ],
        scratch_shapes=[pltpu.VMEM((ROWS, 128), jnp.float32)],
    )(cls3)

    nslots = _SLOTR * 128
    scores_f = scores_r.reshape(B, 1, nslots)
    sel_f = sel_r.reshape(B, 1, nslots)

    tab = jnp.concatenate(
        [boxes3D, locations, translation.reshape(B, N, 45),
         confidence.reshape(B, N, 45)], axis=-1)
    rot = rotation.reshape(B, N, 90)

    smem_spec = pl.BlockSpec((1, 1, nslots), lambda b: (b, 0, 0),
                             memory_space=pltpu.SMEM)

    blo, roto, cfo = pl.pallas_call(
        _gather_kern,
        grid=(B,),
        in_specs=[smem_spec, smem_spec,
                  pl.BlockSpec((1, N, 108), lambda b: (b, 0, 0)),
                  pl.BlockSpec((1, N, 90), lambda b: (b, 0, 0))],
        out_specs=[
            pl.BlockSpec((1, 304, 108), lambda b: (b, 0, 0)),
            pl.BlockSpec((1, 304, 90), lambda b: (b, 0, 0)),
            pl.BlockSpec((1, 304, 15), lambda b: (b, 0, 0)),
        ],
        out_shape=[
            jax.ShapeDtypeStruct((B, 304, 108), jnp.float32),
            jax.ShapeDtypeStruct((B, 304, 90), jnp.float32),
            jax.ShapeDtypeStruct((B, 304, 15), jnp.float32),
        ],
        scratch_shapes=[pltpu.VMEM((304, 45), jnp.float32)],
    )(sel_f, scores_f, tab, rot)

    top_scores = scores_r.reshape(B, nslots)[:, :_K]
    labels = labels_r.reshape(B, nslots)[:, :_K]
    sel_out = sel_r.reshape(B, nslots)[:, :_K]
    b3 = blo[:, :_K, :16]
    loc = blo[:, :_K, 16:18]
    tr_o = blo[:, :_K, 18:63].reshape(B, _K, 15, 3)
    rot_o = roto[:, :_K].reshape(B, _K, 15, 6)
    conf_o = cfo[:, :_K]
    return (b3, loc, top_scores, labels, tr_o, rot_o, conf_o, sel_out)


# speedup vs baseline: 1.1795x; 1.1795x over previous
import jax
import jax.numpy as jnp
from jax import lax
from jax.experimental import pallas as pl
from jax.experimental.pallas import tpu as pltpu

_THR = 0.5
_K = 300
_SLOTR = 3
_NCH = 160
_G = 50


def _topk_kern(x_ref, s_out, lab_out, sel_out, c_ref):
    C = 15
    ROWS = C * _NCH
    GR = ROWS // _G
    B = x_ref.shape[0]
    for b in range(B):
        c_ref[b] = jnp.where(x_ref[b] > _THR, x_ref[b], 0.0)
    cm0 = tuple(
        jnp.concatenate(
            [jnp.max(c_ref[b, g * GR:(g + 1) * GR], axis=0, keepdims=True)
             for g in range(_G)], axis=0)
        for b in range(B))
    slot_i = (lax.broadcasted_iota(jnp.int32, (_SLOTR, 128), 0) * 128
              + lax.broadcasted_iota(jnp.int32, (_SLOTR, 128), 1))
    g_i = lax.broadcasted_iota(jnp.int32, (_G, 128), 0)
    fl_i = (lax.broadcasted_iota(jnp.int32, (GR, 128), 0) * 128
            + lax.broadcasted_iota(jnp.int32, (GR, 128), 1))
    big = jnp.int32(2 ** 30)

    def body(i, car):
        cms, oss, ols, ons = car
        ncms, noss, nols, nons = [], [], [], []
        for b in range(B):
            cm, os_, ol, on = cms[b], oss[b], ols[b], ons[b]
            m = jnp.max(cm)
            valid = m > _THR
            gstar = jnp.min(jnp.where(cm == m, g_i, big))
            start = gstar * GR
            blk = c_ref[b, pl.ds(start, GR), :]
            fl = jnp.min(jnp.where(blk == m, fl_i, big))
            rloc = fl // 128
            lstar = fl - rloc * 128
            nblk = jnp.where(fl_i == fl, 0.0, blk)
            c_ref[b, pl.ds(start, GR), :] = nblk
            nmax = jnp.max(nblk, axis=0, keepdims=True)
            cm = jnp.where(g_i == gstar, nmax, cm)
            r = start + rloc
            cls_idx = r // _NCH
            n = (r - cls_idx * _NCH) * 128 + lstar
            upd = (slot_i == i) & valid
            ncms.append(cm)
            noss.append(jnp.where(upd, m, os_))
            nols.append(jnp.where(upd, cls_idx, ol))
            nons.append(jnp.where(upd, n, on))
        return (tuple(ncms), tuple(noss), tuple(nols), tuple(nons))

    neg1f = jnp.full((_SLOTR, 128), -1.0, jnp.float32)
    neg1i = jnp.full((_SLOTR, 128), -1, jnp.int32)
    init = (cm0, (neg1f,) * B, (neg1i,) * B, (neg1i,) * B)
    _, oss, ols, ons = lax.fori_loop(0, _K, body, init)
    for b in range(B):
        s_out[b] = oss[b]
        lab_out[b] = ols[b]
        sel_out[b] = ons[b]


def _gather_kern(sel_ref, sc_ref, tab_ref, rot_ref, out_ref, roto, cfo, cfs):
    sum3 = ((lax.broadcasted_iota(jnp.int32, (45, 15), 0) // 3)
            == lax.broadcasted_iota(jnp.int32, (45, 15), 1)).astype(jnp.float32)

    def body(r, _):
        n = sel_ref[0, 0, r]
        s = sc_ref[0, 0, r]
        valid = s > _THR
        nsafe = jnp.maximum(n, 0)
        row = tab_ref[0, pl.ds(nsafe, 1), :]
        out_ref[0, pl.ds(r, 1), :] = jnp.where(valid, row, -1.0)
        rrow = rot_ref[0, pl.ds(nsafe, 1), :]
        roto[0, pl.ds(r, 1), :] = jnp.where(valid, rrow, -1.0)
        cfs[pl.ds(r, 1), :] = jnp.where(valid, jnp.abs(row[:, 63:108]), -1.0 / 3.0)
        return 0

    lax.fori_loop(0, 304, body, 0)
    cfo[0] = jnp.dot(cfs[...], sum3, preferred_element_type=jnp.float32)


def kernel(boxes3D, classification, locations, translation, rotation, confidence):
    B, N, C = classification.shape
    ROWS = C * _NCH
    NPAD = _NCH * 128

    cls_t = jnp.transpose(classification, (0, 2, 1))
    cls_t = jnp.pad(cls_t, ((0, 0), (0, 0), (0, NPAD - N)))
    cls3 = cls_t.reshape(B, ROWS, 128)

    scores_r, labels_r, sel_r = pl.pallas_call(
        _topk_kern,
        out_shape=[
            jax.ShapeDtypeStruct((B, _SLOTR, 128), jnp.float32),
            jax.ShapeDtypeStruct((B, _SLOTR, 128), jnp.int32),
            jax.ShapeDtypeStruct((B, _SLOTR, 128), jnp.int32),
        ],
        scratch_shapes=[pltpu.VMEM((B, ROWS, 128), jnp.float32)],
    )(cls3)

    nslots = _SLOTR * 128
    scores_f = scores_r.reshape(B, 1, nslots)
    sel_f = sel_r.reshape(B, 1, nslots)

    tab = jnp.concatenate(
        [boxes3D, locations, translation.reshape(B, N, 45),
         confidence.reshape(B, N, 45)], axis=-1)
    rot = rotation.reshape(B, N, 90)

    smem_spec = pl.BlockSpec((1, 1, nslots), lambda b: (b, 0, 0),
                             memory_space=pltpu.SMEM)

    blo, roto, cfo = pl.pallas_call(
        _gather_kern,
        grid=(B,),
        in_specs=[smem_spec, smem_spec,
                  pl.BlockSpec((1, N, 108), lambda b: (b, 0, 0)),
                  pl.BlockSpec((1, N, 90), lambda b: (b, 0, 0))],
        out_specs=[
            pl.BlockSpec((1, 304, 108), lambda b: (b, 0, 0)),
            pl.BlockSpec((1, 304, 90), lambda b: (b, 0, 0)),
            pl.BlockSpec((1, 304, 15), lambda b: (b, 0, 0)),
        ],
        out_shape=[
            jax.ShapeDtypeStruct((B, 304, 108), jnp.float32),
            jax.ShapeDtypeStruct((B, 304, 90), jnp.float32),
            jax.ShapeDtypeStruct((B, 304, 15), jnp.float32),
        ],
        scratch_shapes=[pltpu.VMEM((304, 45), jnp.float32)],
    )(sel_f, scores_f, tab, rot)

    top_scores = scores_r.reshape(B, nslots)[:, :_K]
    labels = labels_r.reshape(B, nslots)[:, :_K]
    sel_out = sel_r.reshape(B, nslots)[:, :_K]
    b3 = blo[:, :_K, :16]
    loc = blo[:, :_K, 16:18]
    tr_o = blo[:, :_K, 18:63].reshape(B, _K, 15, 3)
    rot_o = roto[:, :_K].reshape(B, _K, 15, 6)
    conf_o = cfo[:, :_K]
    return (b3, loc, top_scores, labels, tr_o, rot_o, conf_o, sel_out)
